# baseline (speedup 1.0000x reference)
import jax
import jax.numpy as jnp
from jax.experimental import pallas as pl

_EPS = 1e-5
_KS = [32, 16, 16]
_N_GRAPHS = 8


def _linear(p, x):
    y = x @ p["W"].T
    if "b" in p:
        y = y + p["b"]
    return y


def _layer_norm(x, g, b):
    mu = jnp.mean(x, axis=-1, keepdims=True)
    var = jnp.mean((x - mu) ** 2, axis=-1, keepdims=True)
    return (x - mu) / jnp.sqrt(var + _EPS) * g + b


def _gravnet_conv(x, bp, cross_mask, k):
    s = _linear(bp["lin_s"], x)
    h = _linear(bp["lin_h"], x)
    sq = jnp.sum(s * s, axis=-1)
    d2 = sq[:, None] + sq[None, :] - 2.0 * (s @ s.T)
    d2 = jnp.maximum(d2, 0.0)
    d2 = jnp.where(cross_mask, jnp.inf, d2)
    neg_d2, idx = jax.lax.top_k(-d2, k)
    w = jnp.exp(10.0 * neg_d2)
    msg = h[idx] * w[..., None]
    mean_agg = jnp.mean(msg, axis=1)
    max_agg = jnp.max(msg, axis=1)
    agg = jnp.concatenate([mean_agg, max_agg], axis=-1)
    return _linear(bp["lin_out1"], x) + _linear(bp["lin_out2"], agg)


def _heads_kernel(pooled_ref, wm_ref, bm_ref, g_ref, b_ref, wc_ref, bc_ref,
                  we_ref, be_ref, cls_ref, en_ref):
    pooled = pooled_ref[...]
    e = jnp.dot(pooled, wm_ref[...].T, preferred_element_type=jnp.float32)
    e = e + bm_ref[...]
    e = jnp.maximum(e, 0.0)
    mu = jnp.mean(e, axis=-1, keepdims=True)
    var = jnp.mean((e - mu) ** 2, axis=-1, keepdims=True)
    e = (e - mu) / jnp.sqrt(var + _EPS) * g_ref[...] + b_ref[...]
    cls_ref[...] = jnp.dot(e, wc_ref[...].T, preferred_element_type=jnp.float32) + bc_ref[...]
    en_ref[...] = jnp.dot(e, we_ref[...].T, preferred_element_type=jnp.float32) + be_ref[...]


def _heads(pooled, params):
    n_graphs = pooled.shape[0]
    wc = jnp.zeros((128, 128), jnp.float32).at[:4].set(params["cls"]["W"])
    bc = jnp.zeros((128,), jnp.float32).at[:4].set(params["cls"]["b"])
    we = jnp.zeros((128, 128), jnp.float32).at[:1].set(params["energy"]["W"])
    be = jnp.zeros((128,), jnp.float32).at[:1].set(params["energy"]["b"])
    cls, en = pl.pallas_call(
        _heads_kernel,
        out_shape=(
            jax.ShapeDtypeStruct((n_graphs, 128), jnp.float32),
            jax.ShapeDtypeStruct((n_graphs, 128), jnp.float32),
        ),
    )(pooled, params["mlp_lin"]["W"], params["mlp_lin"]["b"],
      params["mlp_ln_g"], params["mlp_ln_b"],
      wc, bc, we, be)
    return cls[:, :4], en[:, 0]


def kernel(x, edge_index, batch, params):
    cross_mask = batch[:, None] != batch[None, :]
    h = _linear(params["input_lin"], x)
    for blk, k in zip(params["blocks"], _KS):
        h_in = h
        h2 = _gravnet_conv(h, blk, cross_mask, k)
        h2 = _linear(blk["lin"], h2)
        h2 = _layer_norm(h2, blk["ln_g"], blk["ln_b"])
        h2 = jax.nn.relu(h2)
        h = h2 + h_in
    sums = jax.ops.segment_sum(h, batch, num_segments=_N_GRAPHS)
    counts = jax.ops.segment_sum(jnp.ones((h.shape[0],), jnp.float32), batch,
                                 num_segments=_N_GRAPHS)
    pooled = sums / jnp.maximum(counts, 1.0)[:, None]
    return _heads(pooled, params)


# speedup vs baseline: 4.4730x; 4.4730x over previous
import functools

import jax
import jax.numpy as jnp
from jax.experimental import pallas as pl
from jax.experimental.pallas import tpu as pltpu

_EPS = 1e-5
_KS = [32, 16, 16]
_NG = 8
_R = 128
_C = 512
_NEG = -1e30

_call = pl.pallas_call


def _nt(a, b):
    return jax.lax.dot_general(a, b, (((1,), (1,)), ((), ())),
                               preferred_element_type=jnp.float32)


def _nn(a, b):
    return jax.lax.dot_general(a, b, (((1,), (0,)), ((), ())),
                               preferred_element_type=jnp.float32)


def _mm_kernel(x_ref, w_ref, b_ref, o_ref):
    o_ref[...] = _nt(x_ref[...], w_ref[...]) + b_ref[...]


def _matmul_bias(x, w, b):
    n_pad = x.shape[0]
    nt = n_pad // _R
    return _call(
        _mm_kernel,
        grid=(nt,),
        in_specs=[
            pl.BlockSpec((_R, x.shape[1]), lambda i: (i, 0)),
            pl.BlockSpec(w.shape, lambda i: (0, 0)),
            pl.BlockSpec(b.shape, lambda i: (0, 0)),
        ],
        out_specs=pl.BlockSpec((_R, w.shape[0]), lambda i: (i, 0)),
        out_shape=jax.ShapeDtypeStruct((n_pad, w.shape[0]), jnp.float32),
    )(x, w, b)


def _proj_kernel(h_ref, ws_ref, bs_ref, wh_ref, bh_ref, s_ref, hp_ref):
    h = h_ref[...]
    s_ref[...] = _nt(h, ws_ref[...]) + bs_ref[...]
    hp_ref[...] = _nt(h, wh_ref[...]) + bh_ref[...]


def _proj(h, ws_pad, bs_pad, wh, bh):
    n_pad = h.shape[0]
    nt = n_pad // _R
    return _call(
        _proj_kernel,
        grid=(nt,),
        in_specs=[
            pl.BlockSpec((_R, 128), lambda i: (i, 0)),
            pl.BlockSpec((128, 128), lambda i: (0, 0)),
            pl.BlockSpec((1, 128), lambda i: (0, 0)),
            pl.BlockSpec((32, 128), lambda i: (0, 0)),
            pl.BlockSpec((1, 32), lambda i: (0, 0)),
        ],
        out_specs=(
            pl.BlockSpec((_R, 128), lambda i: (i, 0)),
            pl.BlockSpec((_R, 32), lambda i: (i, 0)),
        ),
        out_shape=(
            jax.ShapeDtypeStruct((n_pad, 128), jnp.float32),
            jax.ShapeDtypeStruct((n_pad, 32), jnp.float32),
        ),
    )(h, ws_pad, bs_pad, wh, bh)


def _gravnet_kernel(lo_ref, hi_ref, h_ref, br_ref, s_ref, hp_ref, bc_ref,
                    w1_ref, w2_ref, b2_ref, wl_ref, bl_ref, g_ref, bb_ref,
                    o_ref, d_scr, *, k, n_pad):
    i = pl.program_id(0)
    lo = lo_ref[i]
    hi = hi_ref[i]
    c0 = lo // _C
    c1 = (hi + _C - 1) // _C

    h = h_ref[...]
    sr = s_ref[pl.ds(i * _R, _R), :]
    sq_r = jnp.sum(sr * sr, axis=1, keepdims=True)
    b_r = br_ref[...]
    ones_row = jnp.ones((1, 128), jnp.float32)

    def body_a(c, _):
        sl = pl.ds(c * _C, _C)
        sc = s_ref[sl, :]
        g = _nt(sr, sc)
        sq_c = _nt(ones_row, sc * sc)
        d2 = jnp.maximum(sq_r + sq_c - 2.0 * g, 0.0)
        bc = bc_ref[0:1, sl]
        d_scr[:, sl] = jnp.where(b_r != bc, _NEG, -d2)
        return 0

    jax.lax.fori_loop(c0, c1, body_a, 0)

    def body_k(_, carry):
        sum_acc, max_acc = carry

        def body_m(c, bc_):
            best, bidx = bc_
            sl = pl.ds(c * _C, _C)
            d = d_scr[:, sl]
            m = jnp.max(d, axis=1, keepdims=True)
            io = jax.lax.broadcasted_iota(jnp.int32, (_R, _C), 1) + c * _C
            fi = jnp.min(jnp.where(d == m, io, n_pad), axis=1, keepdims=True)
            take = (m > best) | ((m == best) & (fi < bidx))
            return (jnp.where(take, m, best), jnp.where(take, fi, bidx))

        best, bidx = jax.lax.fori_loop(
            c0, c1, body_m,
            (jnp.full((_R, 1), -3e38, jnp.float32),
             jnp.full((_R, 1), n_pad, jnp.int32)))
        w = jnp.exp(10.0 * best)

        def body_s(c, hs):
            sl = pl.ds(c * _C, _C)
            d = d_scr[:, sl]
            io = jax.lax.broadcasted_iota(jnp.int32, (_R, _C), 1) + c * _C
            one = io == bidx
            d_scr[:, sl] = jnp.where(one, _NEG, d)
            return hs + _nn(one.astype(jnp.float32), hp_ref[sl, :])

        hsel = jax.lax.fori_loop(c0, c1, body_s,
                                 jnp.zeros((_R, 32), jnp.float32))
        msg = w * hsel
        return (sum_acc + msg, jnp.maximum(max_acc, msg))

    sum_acc, max_acc = jax.lax.fori_loop(
        0, k, body_k,
        (jnp.zeros((_R, 32), jnp.float32),
         jnp.full((_R, 32), -3e38, jnp.float32)))

    agg = jnp.concatenate([sum_acc * (1.0 / k), max_acc], axis=1)
    out = _nt(h, w1_ref[...]) + _nt(agg, w2_ref[...]) + b2_ref[...]
    h2 = _nt(out, wl_ref[...]) + bl_ref[...]
    mu = jnp.mean(h2, axis=1, keepdims=True)
    var = jnp.mean((h2 - mu) ** 2, axis=1, keepdims=True)
    h2 = (h2 - mu) / jnp.sqrt(var + _EPS) * g_ref[...] + bb_ref[...]
    o_ref[...] = jnp.maximum(h2, 0.0) + h


def _gravnet_block(tile_lo, tile_hi, h, batch_row, s_pad, hp, batch_col,
                   w1, w2, b2, wl, bl, g, bb, k):
    n_pad = h.shape[0]
    nt = n_pad // _R
    full = lambda shape: pl.BlockSpec(shape, lambda i, *_: (0, 0))
    grid_spec = pltpu.PrefetchScalarGridSpec(
        num_scalar_prefetch=2,
        grid=(nt,),
        in_specs=[
            pl.BlockSpec((_R, 128), lambda i, *_: (i, 0)),
            pl.BlockSpec((_R, 1), lambda i, *_: (i, 0)),
            full((n_pad, 128)),
            full((n_pad, 32)),
            full((_NG, n_pad)),
            full((128, 128)),
            full((128, 64)),
            full((1, 128)),
            full((128, 128)),
            full((1, 128)),
            full((1, 128)),
            full((1, 128)),
        ],
        out_specs=pl.BlockSpec((_R, 128), lambda i, *_: (i, 0)),
        scratch_shapes=[pltpu.VMEM((_R, n_pad), jnp.float32)],
    )
    return _call(
        functools.partial(_gravnet_kernel, k=k, n_pad=n_pad),
        grid_spec=grid_spec,
        out_shape=jax.ShapeDtypeStruct((n_pad, 128), jnp.float32),
    )(tile_lo, tile_hi, h, batch_row, s_pad, hp, batch_col,
      w1, w2, b2, wl, bl, g, bb)


def _pool_kernel(h_ref, bc_ref, s_ref, c_ref):
    i = pl.program_id(0)

    @pl.when(i == 0)
    def _():
        s_ref[...] = jnp.zeros_like(s_ref)
        c_ref[...] = jnp.zeros_like(c_ref)

    bct = bc_ref[0:1, pl.ds(i * _R, _R)]
    gio = jax.lax.broadcasted_iota(jnp.int32, (_NG, _R), 0)
    one = (gio == bct).astype(jnp.float32)
    s_ref[...] += _nn(one, h_ref[...])
    c_ref[...] += _nn(one, jnp.ones((_R, 128), jnp.float32))


def _pool(h, batch_col):
    n_pad = h.shape[0]
    nt = n_pad // _R
    return _call(
        _pool_kernel,
        grid=(nt,),
        in_specs=[
            pl.BlockSpec((_R, 128), lambda i: (i, 0)),
            pl.BlockSpec((_NG, n_pad), lambda i: (0, 0)),
        ],
        out_specs=(
            pl.BlockSpec((_NG, 128), lambda i: (0, 0)),
            pl.BlockSpec((_NG, 128), lambda i: (0, 0)),
        ),
        out_shape=(
            jax.ShapeDtypeStruct((_NG, 128), jnp.float32),
            jax.ShapeDtypeStruct((_NG, 128), jnp.float32),
        ),
    )(h, batch_col)


def _heads_kernel(s_ref, c_ref, wm_ref, bm_ref, g_ref, b_ref, wc_ref, bc_ref,
                  we_ref, be_ref, cls_ref, en_ref):
    pooled = s_ref[...] / jnp.maximum(c_ref[...], 1.0)
    e = _nt(pooled, wm_ref[...]) + bm_ref[...]
    e = jnp.maximum(e, 0.0)
    mu = jnp.mean(e, axis=-1, keepdims=True)
    var = jnp.mean((e - mu) ** 2, axis=-1, keepdims=True)
    e = (e - mu) / jnp.sqrt(var + _EPS) * g_ref[...] + b_ref[...]
    cls_ref[...] = _nt(e, wc_ref[...]) + bc_ref[...]
    en_ref[...] = _nt(e, we_ref[...]) + be_ref[...]


def _heads(sums, counts, params):
    wc = jnp.zeros((128, 128), jnp.float32).at[:4].set(params["cls"]["W"])
    bc = jnp.zeros((1, 128), jnp.float32).at[0, :4].set(params["cls"]["b"])
    we = jnp.zeros((128, 128), jnp.float32).at[:1].set(params["energy"]["W"])
    be = jnp.zeros((1, 128), jnp.float32).at[0, :1].set(params["energy"]["b"])
    cls, en = _call(
        _heads_kernel,
        out_shape=(
            jax.ShapeDtypeStruct((_NG, 128), jnp.float32),
            jax.ShapeDtypeStruct((_NG, 128), jnp.float32),
        ),
    )(sums, counts, params["mlp_lin"]["W"],
      params["mlp_lin"]["b"].reshape(1, 128),
      params["mlp_ln_g"].reshape(1, 128), params["mlp_ln_b"].reshape(1, 128),
      wc, bc, we, be)
    return cls[:, :4], en[:, 0]


def kernel(x, edge_index, batch, params):
    n = x.shape[0]
    n_pad = pl.cdiv(n, _C) * _C
    nt = n_pad // _R

    xp = jnp.pad(x, ((0, n_pad - n), (0, 0)))
    bp = jnp.pad(batch, (0, n_pad - n), constant_values=_NG)
    batch_row = bp.reshape(n_pad, 1)
    batch_col = jnp.tile(bp[None, :], (_NG, 1))

    offsets = jnp.searchsorted(batch, jnp.arange(_NG + 1)).astype(jnp.int32)
    r0 = jnp.arange(nt, dtype=jnp.int32) * _R
    r1 = jnp.minimum(r0 + _R - 1, n_pad - 1)
    tile_lo = offsets[jnp.clip(bp[r0], 0, _NG - 1)]
    tile_hi = offsets[jnp.clip(bp[r1], 0, _NG - 1) + 1]

    h = _matmul_bias(xp, params["input_lin"]["W"],
                     params["input_lin"]["b"].reshape(1, 128))

    for blk, k in zip(params["blocks"], _KS):
        ws_pad = jnp.zeros((128, 128), jnp.float32).at[:4].set(blk["lin_s"]["W"])
        bs_pad = jnp.zeros((1, 128), jnp.float32).at[0, :4].set(blk["lin_s"]["b"])
        s_pad, hp = _proj(h, ws_pad, bs_pad, blk["lin_h"]["W"],
                          blk["lin_h"]["b"].reshape(1, 32))
        h = _gravnet_block(
            tile_lo, tile_hi, h, batch_row, s_pad, hp, batch_col,
            blk["lin_out1"]["W"], blk["lin_out2"]["W"],
            blk["lin_out2"]["b"].reshape(1, 128),
            blk["lin"]["W"], blk["lin"]["b"].reshape(1, 128),
            blk["ln_g"].reshape(1, 128), blk["ln_b"].reshape(1, 128), k)

    sums, counts = _pool(h, batch_col)
    return _heads(sums, counts, params)


# speedup vs baseline: 4.4800x; 1.0016x over previous
import functools

import jax
import jax.numpy as jnp
from jax.experimental import pallas as pl
from jax.experimental.pallas import tpu as pltpu

_EPS = 1e-5
_KS = [32, 16, 16]
_NG = 8
_R = 128
_C = 512
_NEG = -1e30

_call = pl.pallas_call
_PARALLEL = pltpu.CompilerParams(dimension_semantics=("parallel",))


def _nt(a, b):
    return jax.lax.dot_general(a, b, (((1,), (1,)), ((), ())),
                               preferred_element_type=jnp.float32)


def _nn(a, b):
    return jax.lax.dot_general(a, b, (((1,), (0,)), ((), ())),
                               preferred_element_type=jnp.float32)


def _mm_kernel(x_ref, w_ref, b_ref, o_ref):
    o_ref[...] = _nt(x_ref[...], w_ref[...]) + b_ref[...]


def _matmul_bias(x, w, b):
    n_pad = x.shape[0]
    nt = n_pad // _R
    return _call(
        _mm_kernel,
        grid=(nt,),
        in_specs=[
            pl.BlockSpec((_R, x.shape[1]), lambda i: (i, 0)),
            pl.BlockSpec(w.shape, lambda i: (0, 0)),
            pl.BlockSpec(b.shape, lambda i: (0, 0)),
        ],
        out_specs=pl.BlockSpec((_R, w.shape[0]), lambda i: (i, 0)),
        out_shape=jax.ShapeDtypeStruct((n_pad, w.shape[0]), jnp.float32),
        compiler_params=_PARALLEL,
    )(x, w, b)


def _proj_kernel(h_ref, ws_ref, bs_ref, wh_ref, bh_ref, s_ref, hp_ref):
    h = h_ref[...]
    s_ref[...] = _nt(h, ws_ref[...]) + bs_ref[...]
    hp_ref[...] = _nt(h, wh_ref[...]) + bh_ref[...]


def _proj(h, ws_pad, bs_pad, wh, bh):
    n_pad = h.shape[0]
    nt = n_pad // _R
    return _call(
        _proj_kernel,
        grid=(nt,),
        in_specs=[
            pl.BlockSpec((_R, 128), lambda i: (i, 0)),
            pl.BlockSpec((128, 128), lambda i: (0, 0)),
            pl.BlockSpec((1, 128), lambda i: (0, 0)),
            pl.BlockSpec((32, 128), lambda i: (0, 0)),
            pl.BlockSpec((1, 32), lambda i: (0, 0)),
        ],
        out_specs=(
            pl.BlockSpec((_R, 128), lambda i: (i, 0)),
            pl.BlockSpec((_R, 32), lambda i: (i, 0)),
        ),
        out_shape=(
            jax.ShapeDtypeStruct((n_pad, 128), jnp.float32),
            jax.ShapeDtypeStruct((n_pad, 32), jnp.float32),
        ),
        compiler_params=_PARALLEL,
    )(h, ws_pad, bs_pad, wh, bh)


def _gravnet_kernel(lo_ref, hi_ref, h_ref, br_ref, s_ref, hp_ref, bc_ref,
                    w1_ref, w2_ref, b2_ref, wl_ref, bl_ref, g_ref, bb_ref,
                    o_ref, d_scr, *, k, n_pad):
    i = pl.program_id(0)
    lo = lo_ref[i]
    hi = hi_ref[i]
    c0 = lo // _C
    c1 = (hi + _C - 1) // _C

    h = h_ref[...]
    sr = s_ref[pl.ds(i * _R, _R), :]
    sq_r = jnp.sum(sr * sr, axis=1, keepdims=True)
    b_r = br_ref[...]
    ones_row = jnp.ones((1, 128), jnp.float32)

    def body_a(c, _):
        sl = pl.ds(c * _C, _C)
        sc = s_ref[sl, :]
        g = _nt(sr, sc)
        sq_c = _nt(ones_row, sc * sc)
        d2 = jnp.maximum(sq_r + sq_c - 2.0 * g, 0.0)
        bc = bc_ref[0:1, sl]
        d_scr[:, sl] = jnp.where(b_r != bc, _NEG, -d2)
        return 0

    jax.lax.fori_loop(c0, c1, body_a, 0)

    def body_k(_, carry):
        sum_acc, max_acc = carry

        def body_m(c, bc_):
            best, bidx = bc_
            sl = pl.ds(c * _C, _C)
            d = d_scr[:, sl]
            m = jnp.max(d, axis=1, keepdims=True)
            io = jax.lax.broadcasted_iota(jnp.int32, (_R, _C), 1) + c * _C
            fi = jnp.min(jnp.where(d == m, io, n_pad), axis=1, keepdims=True)
            take = (m > best) | ((m == best) & (fi < bidx))
            return (jnp.where(take, m, best), jnp.where(take, fi, bidx))

        best, bidx = jax.lax.fori_loop(
            c0, c1, body_m,
            (jnp.full((_R, 1), -3e38, jnp.float32),
             jnp.full((_R, 1), n_pad, jnp.int32)))
        w = jnp.exp(10.0 * best)

        def body_s(c, hs):
            sl = pl.ds(c * _C, _C)
            d = d_scr[:, sl]
            io = jax.lax.broadcasted_iota(jnp.int32, (_R, _C), 1) + c * _C
            one = io == bidx
            d_scr[:, sl] = jnp.where(one, _NEG, d)
            return hs + _nn(one.astype(jnp.float32), hp_ref[sl, :])

        hsel = jax.lax.fori_loop(c0, c1, body_s,
                                 jnp.zeros((_R, 32), jnp.float32))
        msg = w * hsel
        return (sum_acc + msg, jnp.maximum(max_acc, msg))

    sum_acc, max_acc = jax.lax.fori_loop(
        0, k, body_k,
        (jnp.zeros((_R, 32), jnp.float32),
         jnp.full((_R, 32), -3e38, jnp.float32)))

    agg = jnp.concatenate([sum_acc * (1.0 / k), max_acc], axis=1)
    out = _nt(h, w1_ref[...]) + _nt(agg, w2_ref[...]) + b2_ref[...]
    h2 = _nt(out, wl_ref[...]) + bl_ref[...]
    mu = jnp.mean(h2, axis=1, keepdims=True)
    var = jnp.mean((h2 - mu) ** 2, axis=1, keepdims=True)
    h2 = (h2 - mu) / jnp.sqrt(var + _EPS) * g_ref[...] + bb_ref[...]
    o_ref[...] = jnp.maximum(h2, 0.0) + h


def _gravnet_block(tile_lo, tile_hi, h, batch_row, s_pad, hp, batch_col,
                   w1, w2, b2, wl, bl, g, bb, k):
    n_pad = h.shape[0]
    nt = n_pad // _R
    full = lambda shape: pl.BlockSpec(shape, lambda i, *_: (0, 0))
    grid_spec = pltpu.PrefetchScalarGridSpec(
        num_scalar_prefetch=2,
        grid=(nt,),
        in_specs=[
            pl.BlockSpec((_R, 128), lambda i, *_: (i, 0)),
            pl.BlockSpec((_R, 1), lambda i, *_: (i, 0)),
            full((n_pad, 128)),
            full((n_pad, 32)),
            full((_NG, n_pad)),
            full((128, 128)),
            full((128, 64)),
            full((1, 128)),
            full((128, 128)),
            full((1, 128)),
            full((1, 128)),
            full((1, 128)),
        ],
        out_specs=pl.BlockSpec((_R, 128), lambda i, *_: (i, 0)),
        scratch_shapes=[pltpu.VMEM((_R, n_pad), jnp.float32)],
    )
    return _call(
        functools.partial(_gravnet_kernel, k=k, n_pad=n_pad),
        grid_spec=grid_spec,
        out_shape=jax.ShapeDtypeStruct((n_pad, 128), jnp.float32),
        compiler_params=_PARALLEL,
    )(tile_lo, tile_hi, h, batch_row, s_pad, hp, batch_col,
      w1, w2, b2, wl, bl, g, bb)


def _pool_kernel(h_ref, bc_ref, s_ref, c_ref):
    i = pl.program_id(0)

    @pl.when(i == 0)
    def _():
        s_ref[...] = jnp.zeros_like(s_ref)
        c_ref[...] = jnp.zeros_like(c_ref)

    bct = bc_ref[0:1, pl.ds(i * _R, _R)]
    gio = jax.lax.broadcasted_iota(jnp.int32, (_NG, _R), 0)
    one = (gio == bct).astype(jnp.float32)
    s_ref[...] += _nn(one, h_ref[...])
    c_ref[...] += _nn(one, jnp.ones((_R, 128), jnp.float32))


def _pool(h, batch_col):
    n_pad = h.shape[0]
    nt = n_pad // _R
    return _call(
        _pool_kernel,
        grid=(nt,),
        in_specs=[
            pl.BlockSpec((_R, 128), lambda i: (i, 0)),
            pl.BlockSpec((_NG, n_pad), lambda i: (0, 0)),
        ],
        out_specs=(
            pl.BlockSpec((_NG, 128), lambda i: (0, 0)),
            pl.BlockSpec((_NG, 128), lambda i: (0, 0)),
        ),
        out_shape=(
            jax.ShapeDtypeStruct((_NG, 128), jnp.float32),
            jax.ShapeDtypeStruct((_NG, 128), jnp.float32),
        ),
    )(h, batch_col)


def _heads_kernel(s_ref, c_ref, wm_ref, bm_ref, g_ref, b_ref, wc_ref, bc_ref,
                  we_ref, be_ref, cls_ref, en_ref):
    pooled = s_ref[...] / jnp.maximum(c_ref[...], 1.0)
    e = _nt(pooled, wm_ref[...]) + bm_ref[...]
    e = jnp.maximum(e, 0.0)
    mu = jnp.mean(e, axis=-1, keepdims=True)
    var = jnp.mean((e - mu) ** 2, axis=-1, keepdims=True)
    e = (e - mu) / jnp.sqrt(var + _EPS) * g_ref[...] + b_ref[...]
    cls_ref[...] = _nt(e, wc_ref[...]) + bc_ref[...]
    en_ref[...] = _nt(e, we_ref[...]) + be_ref[...]


def _heads(sums, counts, params):
    wc = jnp.zeros((128, 128), jnp.float32).at[:4].set(params["cls"]["W"])
    bc = jnp.zeros((1, 128), jnp.float32).at[0, :4].set(params["cls"]["b"])
    we = jnp.zeros((128, 128), jnp.float32).at[:1].set(params["energy"]["W"])
    be = jnp.zeros((1, 128), jnp.float32).at[0, :1].set(params["energy"]["b"])
    cls, en = _call(
        _heads_kernel,
        out_shape=(
            jax.ShapeDtypeStruct((_NG, 128), jnp.float32),
            jax.ShapeDtypeStruct((_NG, 128), jnp.float32),
        ),
    )(sums, counts, params["mlp_lin"]["W"],
      params["mlp_lin"]["b"].reshape(1, 128),
      params["mlp_ln_g"].reshape(1, 128), params["mlp_ln_b"].reshape(1, 128),
      wc, bc, we, be)
    return cls[:, :4], en[:, 0]


def kernel(x, edge_index, batch, params):
    n = x.shape[0]
    n_pad = pl.cdiv(n, _C) * _C
    nt = n_pad // _R

    xp = jnp.pad(x, ((0, n_pad - n), (0, 0)))
    bp = jnp.pad(batch, (0, n_pad - n), constant_values=_NG)
    batch_row = bp.reshape(n_pad, 1)
    batch_col = jnp.tile(bp[None, :], (_NG, 1))

    offsets = jnp.searchsorted(batch, jnp.arange(_NG + 1)).astype(jnp.int32)
    r0 = jnp.arange(nt, dtype=jnp.int32) * _R
    r1 = jnp.minimum(r0 + _R - 1, n_pad - 1)
    tile_lo = offsets[jnp.clip(bp[r0], 0, _NG - 1)]
    tile_hi = offsets[jnp.clip(bp[r1], 0, _NG - 1) + 1]

    h = _matmul_bias(xp, params["input_lin"]["W"],
                     params["input_lin"]["b"].reshape(1, 128))

    for blk, k in zip(params["blocks"], _KS):
        ws_pad = jnp.zeros((128, 128), jnp.float32).at[:4].set(blk["lin_s"]["W"])
        bs_pad = jnp.zeros((1, 128), jnp.float32).at[0, :4].set(blk["lin_s"]["b"])
        s_pad, hp = _proj(h, ws_pad, bs_pad, blk["lin_h"]["W"],
                          blk["lin_h"]["b"].reshape(1, 32))
        h = _gravnet_block(
            tile_lo, tile_hi, h, batch_row, s_pad, hp, batch_col,
            blk["lin_out1"]["W"], blk["lin_out2"]["W"],
            blk["lin_out2"]["b"].reshape(1, 128),
            blk["lin"]["W"], blk["lin"]["b"].reshape(1, 128),
            blk["ln_g"].reshape(1, 128), blk["ln_b"].reshape(1, 128), k)

    sums, counts = _pool(h, batch_col)
    return _heads(sums, counts, params)
